# baseline (speedup 1.0000x reference)
import functools

import jax
import jax.numpy as jnp
from jax import lax
from jax.experimental import pallas as pl
from jax.experimental.pallas import tpu as pltpu
from jax.experimental.pallas import tpu_sc as plsc

_D = 64
_NW = 32
_CHUNK = 128
_TBLK = 1024


@functools.lru_cache(maxsize=None)
def _build_repack(V: int):
    grid = (V + _TBLK - 1) // _TBLK

    def body(x_ref, o_ref):
        o_ref[:, 0:_D] = jnp.swapaxes(x_ref[...], 0, 1)

    return pl.pallas_call(
        body,
        grid=(grid,),
        in_specs=[pl.BlockSpec((_D, _TBLK), lambda i: (0, i))],
        out_specs=pl.BlockSpec((_TBLK, 128), lambda i: (i, 0)),
        out_shape=jax.ShapeDtypeStruct((V, 128), jnp.float32),
    )


@functools.lru_cache(maxsize=None)
def _build(T: int, B: int):
    bpw = B // _NW
    nck = bpw // _CHUNK
    total = T * nck
    nbh = B // _CHUNK
    mesh = plsc.VectorSubcoreMesh(core_axis_name="c", subcore_axis_name="s")

    scratch = dict(
        tidb=pltpu.VMEM((bpw,), jnp.int32),
        gbufs=[pltpu.VMEM((_CHUNK, _D), jnp.float32) for _ in range(2)],
        obufs=[pltpu.VMEM((8, 8, _CHUNK), jnp.float32) for _ in range(2)],
        gsems=[pltpu.SemaphoreType.DMA for _ in range(2)],
        osems=[pltpu.SemaphoreType.DMA for _ in range(2)],
    )

    @functools.partial(
        pl.kernel,
        mesh=mesh,
        out_type=jax.ShapeDtypeStruct((T, 8, nbh, 8, _CHUNK), jnp.float32),
        scratch_types=scratch,
        compiler_params=pltpu.CompilerParams(
            use_tc_tiling_on_sc=False, needs_layout_passes=False
        ),
    )
    def gather_kernel(tid_hbm, tab_hbm, out_hbm, *, tidb,
                      gbufs, obufs, gsems, osems):
        wid = lax.axis_index("s") * 2 + lax.axis_index("c")
        b0w = wid * bpw
        lane = lax.iota(jnp.int32, 16)

        def stage_tid(t):
            pltpu.sync_copy(tid_hbm.at[t, pl.ds(b0w, bpw)], tidb)

        def idx_ref(k):
            return tidb.at[pl.ds(k * _CHUNK, _CHUNK)]

        def start_gather(slot, k):
            pltpu.async_copy(tab_hbm.at[idx_ref(k)], gbufs[slot], gsems[slot])

        def wait_gather(slot, k):
            pltpu.make_async_copy(tab_hbm.at[idx_ref(k)], gbufs[slot],
                                  gsems[slot]).wait()

        def transpose_chunk(slot):
            gbuf, obuf = gbufs[slot], obufs[slot]

            def lgroup(g, carry):
                l_ids = g * 16 + lane
                for d in range(16):
                    rot = (lane + d) & 15
                    roth = lax.shift_right_logical(rot, 3)
                    rotl = rot & 7
                    for cg in range(_D // 16):
                        vals = plsc.load_gather(gbuf, [l_ids, rot + cg * 16])
                        plsc.store_scatter(obuf, [roth + cg * 2, rotl, l_ids],
                                           vals)
                return carry

            lax.fori_loop(0, _CHUNK // 16, lgroup, 0)

        def out_ref(t, k):
            return out_hbm.at[t, :, wid * nck + k, :, :]

        def start_store(slot, t, k):
            pltpu.async_copy(obufs[slot], out_ref(t, k), osems[slot])

        def wait_store(slot, t, k):
            pltpu.make_async_copy(obufs[slot], out_ref(t, k),
                                  osems[slot]).wait()

        stage_tid(0)
        start_gather(0, 0)

        def step(j, slot):
            t = j // nck
            k = j - t * nck
            jn = j + 1
            tn = jn // nck
            kn = jn - tn * nck
            nslot = 1 - slot

            wait_gather(slot, k)

            @pl.when(jn < total)
            def _():
                @pl.when(kn == 0)
                def _():
                    stage_tid(tn)
                @pl.when(jn >= 2)
                def _():
                    jp = jn - 2
                    tp = jp // nck
                    wait_store(nslot, tp, jp - tp * nck)
                start_gather(nslot, kn)

            transpose_chunk(slot)
            start_store(slot, t, k)

        def group(g, carry):
            for b in range(2):
                step(2 * g + b, b)
            return carry

        lax.fori_loop(0, total // 2, group, 0)
        for back in (2, 1):
            j = total - back
            t = j // nck
            wait_store(j & 1, t, j - t * nck)

    return gather_kernel


def kernel(token_ids, weights):
    Bt, T = token_ids.shape
    V = weights.shape[0]
    tid_t = token_ids.T.astype(jnp.int32) * 2
    tab = _build_repack(V)(weights.T)
    tab2 = tab.reshape(2 * V, _D)
    out5 = _build(T, Bt)(tid_t, tab2)
    return out5.transpose(2, 4, 0, 1, 3).reshape(Bt, T, _D)


# speedup vs baseline: 1.2217x; 1.2217x over previous
import functools

import jax
import jax.numpy as jnp
from jax import lax
from jax.experimental import pallas as pl
from jax.experimental.pallas import tpu as pltpu
from jax.experimental.pallas import tpu_sc as plsc

_D = 64
_NW = 32
_CHUNK = 128
_TBLK = 1024


@functools.lru_cache(maxsize=None)
def _build_repack(V: int):
    grid = (V + _TBLK - 1) // _TBLK

    def body(x_ref, o_ref):
        o_ref[:, 0:_D] = jnp.swapaxes(x_ref[...], 0, 1)

    return pl.pallas_call(
        body,
        grid=(grid,),
        in_specs=[pl.BlockSpec((_D, _TBLK), lambda i: (0, i))],
        out_specs=pl.BlockSpec((_TBLK, 128), lambda i: (i, 0)),
        out_shape=jax.ShapeDtypeStruct((V, 128), jnp.float32),
    )


@functools.lru_cache(maxsize=None)
def _build(T: int, B: int):
    bpw = B // _NW
    nck = bpw // _CHUNK
    total = T * nck
    nbh = B // _CHUNK
    mesh = plsc.VectorSubcoreMesh(core_axis_name="c", subcore_axis_name="s")

    scratch = dict(
        tidb=pltpu.VMEM((bpw,), jnp.int32),
        gbufs=[pltpu.VMEM((_CHUNK, _D), jnp.float32) for _ in range(2)],
        obufs=[pltpu.VMEM((8, 8, _CHUNK), jnp.float32) for _ in range(2)],
        gsems=[pltpu.SemaphoreType.DMA for _ in range(2)],
        osems=[pltpu.SemaphoreType.DMA for _ in range(2)],
    )

    @functools.partial(
        pl.kernel,
        mesh=mesh,
        out_type=jax.ShapeDtypeStruct((T, 8, nbh, 8, _CHUNK), jnp.float32),
        scratch_types=scratch,
        compiler_params=pltpu.CompilerParams(
            use_tc_tiling_on_sc=False,
            needs_layout_passes=False,
            disable_bounds_checks=True,
        ),
    )
    def gather_kernel(tid_hbm, tab_hbm, out_hbm, *, tidb,
                      gbufs, obufs, gsems, osems):
        wid = lax.axis_index("s") * 2 + lax.axis_index("c")
        b0w = wid * bpw
        lane = lax.iota(jnp.int32, 16)

        def stage_tid(t):
            pltpu.sync_copy(tid_hbm.at[t, pl.ds(b0w, bpw)], tidb)

        def idx_ref(k):
            return tidb.at[pl.ds(k * _CHUNK, _CHUNK)]

        def start_gather(slot, k):
            pltpu.async_copy(tab_hbm.at[idx_ref(k)], gbufs[slot], gsems[slot])

        def wait_gather(slot, k):
            pltpu.make_async_copy(tab_hbm.at[idx_ref(k)], gbufs[slot],
                                  gsems[slot]).wait()

        def transpose_chunk(slot):
            gbuf, obuf = gbufs[slot], obufs[slot]

            def lgroup(g, carry):
                l_ids = g * 16 + lane
                for d in range(16):
                    rot = (lane + d) & 15
                    roth = lax.shift_right_logical(rot, 3)
                    rotl = rot & 7
                    for cg in range(_D // 16):
                        vals = plsc.load_gather(gbuf, [l_ids, rot + cg * 16])
                        plsc.store_scatter(obuf, [roth + cg * 2, rotl, l_ids],
                                           vals)
                return carry

            lax.fori_loop(0, _CHUNK // 16, lgroup, 0)

        def out_ref(t, k):
            return out_hbm.at[t, :, wid * nck + k, :, :]

        def start_store(slot, t, k):
            pltpu.async_copy(obufs[slot], out_ref(t, k), osems[slot])

        def wait_store(slot, t, k):
            pltpu.make_async_copy(obufs[slot], out_ref(t, k),
                                  osems[slot]).wait()

        stage_tid(0)
        start_gather(0, 0)

        def step(j, slot):
            t = j // nck
            k = j - t * nck
            jn = j + 1
            tn = jn // nck
            kn = jn - tn * nck
            nslot = 1 - slot

            wait_gather(slot, k)

            @pl.when(jn < total)
            def _():
                @pl.when(kn == 0)
                def _():
                    stage_tid(tn)
                @pl.when(jn >= 2)
                def _():
                    jp = jn - 2
                    tp = jp // nck
                    wait_store(nslot, tp, jp - tp * nck)
                start_gather(nslot, kn)

            transpose_chunk(slot)
            start_store(slot, t, k)

        def group(g, carry):
            for b in range(2):
                step(2 * g + b, b)
            return carry

        lax.fori_loop(0, total // 2, group, 0)
        for back in (2, 1):
            j = total - back
            t = j // nck
            wait_store(j & 1, t, j - t * nck)

    return gather_kernel


def kernel(token_ids, weights):
    Bt, T = token_ids.shape
    V = weights.shape[0]
    tid_t = token_ids.T.astype(jnp.int32) * 2
    tab = jnp.pad(weights, ((0, 0), (0, 128 - weights.shape[1])))
    tab2 = tab.reshape(2 * V, _D)
    out5 = _build(T, Bt)(tid_t, tab2)
    return out5.transpose(2, 4, 0, 1, 3).reshape(Bt, T, _D)


# speedup vs baseline: 1.3380x; 1.0952x over previous
import functools

import jax
import jax.numpy as jnp
from jax import lax
from jax.experimental import pallas as pl
from jax.experimental.pallas import tpu as pltpu
from jax.experimental.pallas import tpu_sc as plsc

_D = 64
_NW = 32
_CHUNK = 128
_TBLK = 1024


@functools.lru_cache(maxsize=None)
def _build_repack(V: int):
    grid = (V + _TBLK - 1) // _TBLK

    def body(x_ref, o_ref):
        o_ref[:, 0:_D] = jnp.swapaxes(x_ref[...], 0, 1)

    return pl.pallas_call(
        body,
        grid=(grid,),
        in_specs=[pl.BlockSpec((_D, _TBLK), lambda i: (0, i))],
        out_specs=pl.BlockSpec((_TBLK, 128), lambda i: (i, 0)),
        out_shape=jax.ShapeDtypeStruct((V, 128), jnp.float32),
    )


@functools.lru_cache(maxsize=None)
def _build(T: int, B: int):
    bpw = B // _NW
    nck = bpw // _CHUNK
    total = T * nck
    nbh = B // _CHUNK
    mesh = plsc.VectorSubcoreMesh(core_axis_name="c", subcore_axis_name="s")

    scratch = dict(
        tidb=pltpu.VMEM((bpw,), jnp.int32),
        gbufs=[pltpu.VMEM((_CHUNK, _D), jnp.float32) for _ in range(2)],
        obufs=[pltpu.VMEM((8, 8, _CHUNK), jnp.float32) for _ in range(2)],
        gsems=[pltpu.SemaphoreType.DMA for _ in range(2)],
        osems=[pltpu.SemaphoreType.DMA for _ in range(2)],
    )

    @functools.partial(
        pl.kernel,
        mesh=mesh,
        out_type=jax.ShapeDtypeStruct((T, 8, nbh, 8, _CHUNK), jnp.float32),
        scratch_types=scratch,
        compiler_params=pltpu.CompilerParams(
            use_tc_tiling_on_sc=False,
            needs_layout_passes=False,
            disable_bounds_checks=True,
        ),
    )
    def gather_kernel(tid_hbm, tab_hbm, out_hbm, *, tidb,
                      gbufs, obufs, gsems, osems):
        wid = lax.axis_index("s") * 2 + lax.axis_index("c")
        b0w = wid * bpw
        lane = lax.iota(jnp.int32, 16)

        def stage_tid(t):
            pltpu.sync_copy(tid_hbm.at[t, pl.ds(b0w, bpw)], tidb)

        def idx_ref(k):
            return tidb.at[pl.ds(k * _CHUNK, _CHUNK)]

        def start_gather(slot, k):
            pltpu.async_copy(tab_hbm.at[idx_ref(k)], gbufs[slot], gsems[slot])

        def wait_gather(slot, k):
            pltpu.make_async_copy(tab_hbm.at[idx_ref(k)], gbufs[slot],
                                  gsems[slot]).wait()

        def transpose_chunk(slot):
            gbuf, obuf = gbufs[slot], obufs[slot]

            @plsc.parallel_loop(0, _CHUNK // 16, 1)
            def lgroup(g):
                l_ids = g * 16 + lane
                for d in range(16):
                    rot = (lane + d) & 15
                    roth = lax.shift_right_logical(rot, 3)
                    rotl = rot & 7
                    for cg in range(_D // 16):
                        vals = plsc.load_gather(gbuf, [l_ids, rot + cg * 16])
                        plsc.store_scatter(obuf, [roth + cg * 2, rotl, l_ids],
                                           vals)

        def out_ref(t, k):
            return out_hbm.at[t, :, wid * nck + k, :, :]

        def start_store(slot, t, k):
            pltpu.async_copy(obufs[slot], out_ref(t, k), osems[slot])

        def wait_store(slot, t, k):
            pltpu.make_async_copy(obufs[slot], out_ref(t, k),
                                  osems[slot]).wait()

        stage_tid(0)
        start_gather(0, 0)

        def step(j, slot):
            t = j // nck
            k = j - t * nck
            jn = j + 1
            tn = jn // nck
            kn = jn - tn * nck
            nslot = 1 - slot

            wait_gather(slot, k)

            @pl.when(jn < total)
            def _():
                @pl.when(kn == 0)
                def _():
                    stage_tid(tn)
                @pl.when(jn >= 2)
                def _():
                    jp = jn - 2
                    tp = jp // nck
                    wait_store(nslot, tp, jp - tp * nck)
                start_gather(nslot, kn)

            transpose_chunk(slot)
            start_store(slot, t, k)

        def group(g, carry):
            for b in range(2):
                step(2 * g + b, b)
            return carry

        lax.fori_loop(0, total // 2, group, 0)
        for back in (2, 1):
            j = total - back
            t = j // nck
            wait_store(j & 1, t, j - t * nck)

    return gather_kernel


def kernel(token_ids, weights):
    Bt, T = token_ids.shape
    V = weights.shape[0]
    tid_t = token_ids.T.astype(jnp.int32) * 2
    tab = jnp.pad(weights, ((0, 0), (0, 128 - weights.shape[1])))
    tab2 = tab.reshape(2 * V, _D)
    out5 = _build(T, Bt)(tid_t, tab2)
    return out5.transpose(2, 4, 0, 1, 3).reshape(Bt, T, _D)
